# baseline (speedup 1.0000x reference)
import jax
import jax.numpy as jnp
from jax.experimental import pallas as pl

_TH = 64


def _body(v_ref, o_ref):
    d = v_ref.shape[1]
    a = v_ref[0, 0]
    b = v_ref[0, 1]
    m1 = jnp.minimum(a, b)
    m2 = jnp.maximum(a, b)
    for i in range(2, d):
        x = v_ref[0, i]
        m2 = jnp.minimum(m2, jnp.maximum(m1, x))
        m1 = jnp.minimum(m1, x)
    o_ref[0, 0] = m2 - m1


def kernel(volume):
    b, d, h, w = volume.shape
    grid = (b, h // _TH)
    return pl.pallas_call(
        _body,
        grid=grid,
        in_specs=[pl.BlockSpec((1, d, _TH, w), lambda i, j: (i, 0, j, 0))],
        out_specs=pl.BlockSpec((1, 1, _TH, w), lambda i, j: (i, 0, j, 0)),
        out_shape=jax.ShapeDtypeStruct((b, 1, h, w), volume.dtype),
    )(volume)
